# baseline (speedup 1.0000x reference)
import functools
import jax
import jax.numpy as jnp
from jax import lax
from jax.experimental import pallas as pl
from jax.experimental.pallas import tpu as pltpu
from jax.experimental.pallas import tpu_sc as plsc

F32 = jnp.float32
I32 = jnp.int32

D = 128
DEXT = 144
NC, NS, L = 2, 16, 16
NW = NC * NS
B = 96
HPAD = 10240
APAD = 10112
SPLIT0 = 0.375
ATN = 10016


def _round_up(a, m):
    return (a + m - 1) // m * m


def _pre_body(n, x_ref, w_ref, as_ref, ad_ref, hext_ref, adt_ref):
    h = jnp.dot(x_ref[...], w_ref[...].T, preferred_element_type=F32)
    a_s = jnp.sum(h * as_ref[...], axis=1, keepdims=True)
    a_d = jnp.sum(h * ad_ref[...], axis=1)
    hext_ref[pl.ds(0, n), pl.ds(0, D)] = h
    col16 = lax.broadcasted_iota(I32, (n, 16), 1)
    tail = jnp.where(col16 == 0, 1.0, jnp.where(col16 == 1, a_s, 0.0))
    hext_ref[pl.ds(0, n), pl.ds(D, 16)] = tail.astype(F32)
    hext_ref[pl.ds(n, HPAD - n), :] = jnp.zeros((HPAD - n, DEXT), F32)
    adt_ref[0, pl.ds(0, n)] = a_d
    adt_ref[0, pl.ds(n, HPAD - n)] = jnp.zeros((HPAD - n,), F32)


def _tc_pre(x, W, a_src, a_dst):
    n = x.shape[0]
    return pl.pallas_call(
        functools.partial(_pre_body, n),
        out_shape=(
            jax.ShapeDtypeStruct((HPAD, DEXT), F32),
            jax.ShapeDtypeStruct((1, HPAD), F32),
        ),
    )(x, W, a_src.reshape(1, D), a_dst.reshape(1, D))


def _sc_body(nb0, nb1, n, hext, adt, sds, out, acc, idx, hrows, adtile, wbuf,
             ghsem, ssem):
    cid = lax.axis_index("c")
    sid = lax.axis_index("s")
    wid = sid * NC + cid
    nbv = jnp.where(cid == 0, nb0, nb1)
    rpt = APAD // NS
    base = sid * rpt

    pltpu.sync_copy(adt.at[pl.ds(0, ATN)], adtile.at[pl.ds(16, ATN)])

    pltpu.sync_copy(hext.at[pl.ds(HPAD - B, B)], hrows.at[0])
    nfull, rem = rpt // B, rpt % B
    for j in range(nfull):
        pltpu.sync_copy(hrows.at[0], acc.at[pl.ds(base + j * B, B)])
    if rem:
        pltpu.sync_copy(hrows.at[0, pl.ds(0, rem)],
                        acc.at[pl.ds(base + nfull * B, rem)])
    plsc.subcore_barrier()

    def fetch(jb, slot):
        pltpu.sync_copy(sds.at[wid, jb], idx.at[slot])
        pltpu.async_copy(hext.at[idx.at[slot, 0]], hrows.at[slot], ghsem)

    def wait_gather(slot):
        pltpu.make_async_copy(hext.at[idx.at[slot, 0]], hrows.at[slot],
                              ghsem).wait()

    def scatter(slot):
        pltpu.async_copy(hrows.at[slot], acc.at[idx.at[slot, 1]], ssem,
                         add=True)

    def wait_scatter(slot):
        pltpu.make_async_copy(hrows.at[slot], acc.at[idx.at[slot, 1]],
                              ssem).wait()

    def compute(slot):
        for g in range(B // L):
            rows = lax.iota(I32, L) + g * L
            asv = plsc.load_gather(hrows,
                                   [jnp.full((L,), slot, I32), rows,
                                    jnp.full((L,), D + 1, I32)])
            dstv = idx[slot, 1, pl.ds(g * L, L)]
            adv = plsc.load_gather(adtile, [dstv + 16])
            s = asv + adv
            wbuf[pl.ds(L, L)] = jnp.exp(jnp.maximum(s, 0.2 * s))
            for r in range(L):
                wr = plsc.load_gather(wbuf, [jnp.full((L,), L + r, I32)])
                e = g * L + r
                for k in range(DEXT // L):
                    hrows[slot, e, pl.ds(k * L, L)] = (
                        hrows[slot, e, pl.ds(k * L, L)] * wr)

    fetch(0, 0)

    def body(i, carry):
        jb = 2 * i
        wait_gather(0)

        @pl.when(i > 0)
        def _():
            wait_scatter(1)

        fetch(jb + 1, 1)
        compute(0)
        scatter(0)
        wait_gather(1)
        wait_scatter(0)

        @pl.when(jb + 2 < nbv)
        def _():
            fetch(jb + 2, 0)

        compute(1)
        scatter(1)
        return carry

    lax.fori_loop(0, nbv // 2, body, 0)
    wait_scatter(1)
    plsc.subcore_barrier()

    pltpu.sync_copy(acc.at[pl.ds(base, rpt)], out.at[cid, pl.ds(base, rpt)])


def _sc_edge(hext, adt, sds, nb0, nb1, n):
    mesh = plsc.VectorSubcoreMesh(
        core_axis_name="c", subcore_axis_name="s", num_cores=NC,
        num_subcores=NS)
    return pl.kernel(
        functools.partial(_sc_body, nb0, nb1, n),
        out_type=jax.ShapeDtypeStruct((NC, APAD, DEXT), F32),
        mesh=mesh,
        compiler_params=pltpu.CompilerParams(
            use_tc_tiling_on_sc=False, needs_layout_passes=False),
        scratch_types=[
            pltpu.VMEM_SHARED((APAD, DEXT), F32),
            pltpu.VMEM((2, 2, B), I32),
            pltpu.VMEM((2, B, DEXT), F32),
            pltpu.VMEM((16 + ATN,), F32),
            pltpu.VMEM((2 * L,), F32),
            pltpu.SemaphoreType.DMA,
            pltpu.SemaphoreType.DMA,
        ],
    )(hext, adt, sds)


def _post_body(n, x_ref, hext_ref, acc_ref, as_ref, ad_ref, b_ref, g_ref,
               beta_ref, out_ref):
    x = x_ref[...]
    h = hext_ref[pl.ds(0, n), pl.ds(0, D)]
    num = (acc_ref[0, pl.ds(0, n), pl.ds(0, D)]
           + acc_ref[1, pl.ds(0, n), pl.ds(0, D)])
    dent = (acc_ref[0, pl.ds(0, n), pl.ds(D, 16)]
            + acc_ref[1, pl.ds(0, n), pl.ds(D, 16)])
    den = dent[:, 0:1]
    a_s = jnp.sum(h * as_ref[...], axis=1, keepdims=True)
    a_d = jnp.sum(h * ad_ref[...], axis=1, keepdims=True)
    s = a_s + a_d
    w_self = jnp.exp(jnp.maximum(s, 0.2 * s))
    x_att = (num + w_self * h) / (den + w_self + 1e-16) + b_ref[...]
    x2 = x + x_att
    mu = jnp.mean(x2, axis=1, keepdims=True)
    var = jnp.mean((x2 - mu) ** 2, axis=1, keepdims=True)
    xn = (x2 - mu) * lax.rsqrt(var + 1e-5) * g_ref[...] + beta_ref[...]
    out_ref[...] = jnp.maximum(xn, 0.0)


def _tc_post(x, hext, acc, a_src, a_dst, b, g, beta):
    n = x.shape[0]
    return pl.pallas_call(
        functools.partial(_post_body, n),
        out_shape=jax.ShapeDtypeStruct((n, D), F32),
    )(x, hext, acc, a_src.reshape(1, D), a_dst.reshape(1, D),
      b.reshape(1, D), g.reshape(1, D), beta.reshape(1, D))


def kernel(x, edge_index, W0, a_src0, a_dst0, b0, g0, beta0,
           W1, a_src1, a_dst1, b1, g1, beta1):
    n = x.shape[0]
    e = edge_index.shape[1]
    nbsum = max(4, _round_up(-(-e // (NS * B)), 2))
    nb0 = max(2, _round_up(int(nbsum * SPLIT0), 2))
    nb1 = _round_up(nbsum - nb0, 2)
    nbmax = max(nb0, nb1)

    src = edge_index[0].astype(I32)
    dst = edge_index[1].astype(I32)
    cap = NS * (nb0 + nb1) * B
    srcp = jnp.concatenate([src, jnp.full((cap - e,), n, I32)])
    dstp = jnp.concatenate([dst, jnp.full((cap - e,), n, I32)])
    nbw = [nb0 if w % NC == 0 else nb1 for w in range(NW)]
    offs = [0]
    for c in nbw:
        offs.append(offs[-1] + c * B)
    slabs = []
    for w in range(NW):
        cnt = nbw[w] * B
        tailpad = jnp.full(((nbmax - nbw[w]) * B,), n, I32)
        seg_s = jnp.concatenate([srcp[offs[w]:offs[w] + cnt], tailpad])
        seg_d = jnp.concatenate([dstp[offs[w]:offs[w] + cnt], tailpad])
        slabs.append(jnp.stack([seg_s.reshape(nbmax, B),
                                seg_d.reshape(nbmax, B)], axis=1))
    sds = jnp.stack(slabs)

    for (W, a_s, a_d, b, g, beta) in (
            (W0, a_src0, a_dst0, b0, g0, beta0),
            (W1, a_src1, a_dst1, b1, g1, beta1)):
        hext, adt = _tc_pre(x, W, a_s, a_d)
        acc = _sc_edge(hext, adt.reshape(HPAD), sds, nb0, nb1, n)
        x = _tc_post(x, hext, acc, a_s, a_d, b, g, beta)
    return x


# speedup vs baseline: 1.0228x; 1.0228x over previous
import functools
import jax
import jax.numpy as jnp
from jax import lax
from jax.experimental import pallas as pl
from jax.experimental.pallas import tpu as pltpu
from jax.experimental.pallas import tpu_sc as plsc

F32 = jnp.float32
I32 = jnp.int32

D = 128
DEXT = 144
NC, NS, L = 2, 16, 16
NW = NC * NS
B = 80
NSLOT = 3
HPAD = 10240
APAD = 10112
SPLIT0 = 0.375


def _round_up(a, m):
    return (a + m - 1) // m * m


def _pre_body(n, x_ref, w_ref, as_ref, ad_ref, hext_ref, adt_ref):
    h = jnp.dot(x_ref[...], w_ref[...].T, preferred_element_type=F32)
    a_s = jnp.sum(h * as_ref[...], axis=1, keepdims=True)
    a_d = jnp.sum(h * ad_ref[...], axis=1)
    hext_ref[pl.ds(0, n), pl.ds(0, D)] = h
    col16 = lax.broadcasted_iota(I32, (n, 16), 1)
    tail = jnp.where(col16 == 0, 1.0, jnp.where(col16 == 1, a_s, 0.0))
    hext_ref[pl.ds(0, n), pl.ds(D, 16)] = tail.astype(F32)
    hext_ref[pl.ds(n, HPAD - n), :] = jnp.zeros((HPAD - n, DEXT), F32)
    adt_ref[0, pl.ds(0, n)] = a_d
    adt_ref[0, pl.ds(n, HPAD - n)] = jnp.zeros((HPAD - n,), F32)


def _tc_pre(x, W, a_src, a_dst):
    n = x.shape[0]
    return pl.pallas_call(
        functools.partial(_pre_body, n),
        out_shape=(
            jax.ShapeDtypeStruct((HPAD, DEXT), F32),
            jax.ShapeDtypeStruct((1, HPAD), F32),
        ),
    )(x, W, a_src.reshape(1, D), a_dst.reshape(1, D))


def _sc_body(nb0, nb1, n, hext, adt, sds, out, acc, idx, hrows, adbuf, wbuf,
             ghsem, gasem, ssem):
    cid = lax.axis_index("c")
    sid = lax.axis_index("s")
    wid = sid * NC + cid
    nbv = jnp.where(cid == 0, nb0, nb1)
    rpt = APAD // NS
    base = sid * rpt

    pltpu.sync_copy(hext.at[pl.ds(HPAD - B, B)], hrows.at[0])
    nfull, rem = rpt // B, rpt % B
    for j in range(nfull):
        pltpu.sync_copy(hrows.at[0], acc.at[pl.ds(base + j * B, B)])
    if rem:
        pltpu.sync_copy(hrows.at[0, pl.ds(0, rem)],
                        acc.at[pl.ds(base + nfull * B, rem)])
    plsc.subcore_barrier()

    def fetch(jb, slot):
        pltpu.sync_copy(sds.at[wid, jb], idx.at[slot])
        pltpu.async_copy(hext.at[idx.at[slot, 0]], hrows.at[slot], ghsem)
        pltpu.async_copy(adt.at[idx.at[slot, 1]], adbuf.at[slot], gasem)

    def wait_gather(slot):
        pltpu.make_async_copy(hext.at[idx.at[slot, 0]], hrows.at[slot],
                              ghsem).wait()
        pltpu.make_async_copy(adt.at[idx.at[slot, 1]], adbuf.at[slot],
                              gasem).wait()

    def scatter(slot):
        pltpu.async_copy(hrows.at[slot], acc.at[idx.at[slot, 1]], ssem,
                         add=True)

    def wait_scatter(slot):
        pltpu.make_async_copy(hrows.at[slot], acc.at[idx.at[slot, 1]],
                              ssem).wait()

    def compute(slot):
        for g in range(B // L):
            rows = lax.iota(I32, L) + g * L
            asv = plsc.load_gather(hrows,
                                   [jnp.full((L,), slot, I32), rows,
                                    jnp.full((L,), D + 1, I32)])
            adv = adbuf[slot, pl.ds(g * L, L)]
            s = asv + adv
            wbuf[pl.ds(L, L)] = jnp.exp(jnp.maximum(s, 0.2 * s))
            for r in range(L):
                wr = plsc.load_gather(wbuf, [jnp.full((L,), L + r, I32)])
                e = g * L + r
                for k in range(DEXT // L):
                    hrows[slot, e, pl.ds(k * L, L)] = (
                        hrows[slot, e, pl.ds(k * L, L)] * wr)

    fetch(0, 0)
    fetch(1, 1)

    def body(i, carry):
        for u in range(NSLOT):
            jb = NSLOT * i + u
            p = u
            q = (u + 2) % NSLOT
            wait_gather(p)
            compute(p)
            scatter(p)
            if u == 0:
                @pl.when(i > 0)
                def _():
                    wait_scatter(q)
            else:
                wait_scatter(q)

            @pl.when(jb + 2 < nbv)
            def _():
                fetch(jb + 2, q)
        return carry

    lax.fori_loop(0, nbv // NSLOT, body, 0)
    wait_scatter(NSLOT - 1)
    plsc.subcore_barrier()

    pltpu.sync_copy(acc.at[pl.ds(base, rpt)], out.at[cid, pl.ds(base, rpt)])


def _sc_edge(hext, adt, sds, nb0, nb1, n):
    mesh = plsc.VectorSubcoreMesh(
        core_axis_name="c", subcore_axis_name="s", num_cores=NC,
        num_subcores=NS)
    return pl.kernel(
        functools.partial(_sc_body, nb0, nb1, n),
        out_type=jax.ShapeDtypeStruct((NC, APAD, DEXT), F32),
        mesh=mesh,
        compiler_params=pltpu.CompilerParams(
            use_tc_tiling_on_sc=False, needs_layout_passes=False),
        scratch_types=[
            pltpu.VMEM_SHARED((APAD, DEXT), F32),
            pltpu.VMEM((NSLOT, 2, B), I32),
            pltpu.VMEM((NSLOT, B, DEXT), F32),
            pltpu.VMEM((NSLOT, B), F32),
            pltpu.VMEM((2 * L,), F32),
            pltpu.SemaphoreType.DMA,
            pltpu.SemaphoreType.DMA,
            pltpu.SemaphoreType.DMA,
        ],
    )(hext, adt, sds)


def _post_body(n, x_ref, hext_ref, acc_ref, as_ref, ad_ref, b_ref, g_ref,
               beta_ref, out_ref):
    x = x_ref[...]
    h = hext_ref[pl.ds(0, n), pl.ds(0, D)]
    num = (acc_ref[0, pl.ds(0, n), pl.ds(0, D)]
           + acc_ref[1, pl.ds(0, n), pl.ds(0, D)])
    dent = (acc_ref[0, pl.ds(0, n), pl.ds(D, 16)]
            + acc_ref[1, pl.ds(0, n), pl.ds(D, 16)])
    den = dent[:, 0:1]
    a_s = jnp.sum(h * as_ref[...], axis=1, keepdims=True)
    a_d = jnp.sum(h * ad_ref[...], axis=1, keepdims=True)
    s = a_s + a_d
    w_self = jnp.exp(jnp.maximum(s, 0.2 * s))
    x_att = (num + w_self * h) / (den + w_self + 1e-16) + b_ref[...]
    x2 = x + x_att
    mu = jnp.mean(x2, axis=1, keepdims=True)
    var = jnp.mean((x2 - mu) ** 2, axis=1, keepdims=True)
    xn = (x2 - mu) * lax.rsqrt(var + 1e-5) * g_ref[...] + beta_ref[...]
    out_ref[...] = jnp.maximum(xn, 0.0)


def _tc_post(x, hext, acc, a_src, a_dst, b, g, beta):
    n = x.shape[0]
    return pl.pallas_call(
        functools.partial(_post_body, n),
        out_shape=jax.ShapeDtypeStruct((n, D), F32),
    )(x, hext, acc, a_src.reshape(1, D), a_dst.reshape(1, D),
      b.reshape(1, D), g.reshape(1, D), beta.reshape(1, D))


def kernel(x, edge_index, W0, a_src0, a_dst0, b0, g0, beta0,
           W1, a_src1, a_dst1, b1, g1, beta1):
    n = x.shape[0]
    e = edge_index.shape[1]
    nbsum = max(2 * NSLOT, _round_up(-(-e // (NS * B)), NSLOT))
    nb0 = max(NSLOT, _round_up(int(nbsum * SPLIT0), NSLOT))
    nb1 = _round_up(nbsum - nb0, NSLOT)
    nbmax = max(nb0, nb1)

    src = edge_index[0].astype(I32)
    dst = edge_index[1].astype(I32)
    cap = NS * (nb0 + nb1) * B
    srcp = jnp.concatenate([src, jnp.full((cap - e,), n, I32)])
    dstp = jnp.concatenate([dst, jnp.full((cap - e,), n, I32)])
    nbw = [nb0 if w % NC == 0 else nb1 for w in range(NW)]
    offs = [0]
    for c in nbw:
        offs.append(offs[-1] + c * B)
    slabs = []
    for w in range(NW):
        cnt = nbw[w] * B
        tailpad = jnp.full(((nbmax - nbw[w]) * B,), n, I32)
        seg_s = jnp.concatenate([srcp[offs[w]:offs[w] + cnt], tailpad])
        seg_d = jnp.concatenate([dstp[offs[w]:offs[w] + cnt], tailpad])
        slabs.append(jnp.stack([seg_s.reshape(nbmax, B),
                                seg_d.reshape(nbmax, B)], axis=1))
    sds = jnp.stack(slabs)

    for (W, a_s, a_d, b, g, beta) in (
            (W0, a_src0, a_dst0, b0, g0, beta0),
            (W1, a_src1, a_dst1, b1, g1, beta1)):
        hext, adt = _tc_pre(x, W, a_s, a_d)
        acc = _sc_edge(hext, adt.reshape(HPAD), sds, nb0, nb1, n)
        x = _tc_post(x, hext, acc, a_s, a_d, b, g, beta)
    return x


# speedup vs baseline: 1.2379x; 1.2104x over previous
import functools
import jax
import jax.numpy as jnp
from jax import lax
from jax.experimental import pallas as pl
from jax.experimental.pallas import tpu as pltpu
from jax.experimental.pallas import tpu_sc as plsc

F32 = jnp.float32
I32 = jnp.int32

D = 128
DEXT = 144
NC, NS, L = 2, 16, 16
NW = NC * NS
B = 80
NSLOT = 3
HPAD = 10240
APAD = 10112
SPLIT0 = 0.5


def _round_up(a, m):
    return (a + m - 1) // m * m


def _pre_body(n, x_ref, w_ref, as_ref, ad_ref, hext_ref, adt_ref):
    h = jnp.dot(x_ref[...], w_ref[...].T, preferred_element_type=F32)
    a_s = jnp.sum(h * as_ref[...], axis=1, keepdims=True)
    a_d = jnp.sum(h * ad_ref[...], axis=1)
    hext_ref[pl.ds(0, n), pl.ds(0, D)] = h
    col16 = lax.broadcasted_iota(I32, (n, 16), 1)
    tail = jnp.where(col16 == 0, 1.0, jnp.where(col16 == 1, a_s, 0.0))
    hext_ref[pl.ds(0, n), pl.ds(D, 16)] = tail.astype(F32)
    hext_ref[pl.ds(n, HPAD - n), :] = jnp.zeros((HPAD - n, DEXT), F32)
    adt_ref[0, pl.ds(0, n)] = a_d
    adt_ref[0, pl.ds(n, HPAD - n)] = jnp.zeros((HPAD - n,), F32)


def _tc_pre(x, W, a_src, a_dst):
    n = x.shape[0]
    return pl.pallas_call(
        functools.partial(_pre_body, n),
        out_shape=(
            jax.ShapeDtypeStruct((HPAD, DEXT), F32),
            jax.ShapeDtypeStruct((1, HPAD), F32),
        ),
    )(x, W, a_src.reshape(1, D), a_dst.reshape(1, D))


def _sc_body(nb0, nb1, n, hext, adt, sds, out, acc, idx, hrows, adbuf, wbuf,
             ghsem, gasem, ssem):
    cid = lax.axis_index("c")
    sid = lax.axis_index("s")
    wid = sid * NC + cid
    nbv = jnp.where(cid == 0, nb0, nb1)
    rpt = APAD // NS
    base = sid * rpt

    pltpu.sync_copy(hext.at[pl.ds(HPAD - B, B)], hrows.at[0])
    nfull, rem = rpt // B, rpt % B
    for j in range(nfull):
        pltpu.sync_copy(hrows.at[0], acc.at[pl.ds(base + j * B, B)])
    if rem:
        pltpu.sync_copy(hrows.at[0, pl.ds(0, rem)],
                        acc.at[pl.ds(base + nfull * B, rem)])
    plsc.subcore_barrier()

    def fetch(jb, slot):
        pltpu.sync_copy(sds.at[wid, jb], idx.at[slot])
        pltpu.async_copy(hext.at[idx.at[slot, 0]], hrows.at[slot], ghsem)
        pltpu.async_copy(adt.at[idx.at[slot, 1]], adbuf.at[slot], gasem)

    def wait_gather(slot):
        pltpu.make_async_copy(hext.at[idx.at[slot, 0]], hrows.at[slot],
                              ghsem).wait()
        pltpu.make_async_copy(adt.at[idx.at[slot, 1]], adbuf.at[slot],
                              gasem).wait()

    def scatter(slot):
        pltpu.async_copy(hrows.at[slot], acc.at[idx.at[slot, 1]], ssem,
                         add=True)

    def wait_scatter(slot):
        pltpu.make_async_copy(hrows.at[slot], acc.at[idx.at[slot, 1]],
                              ssem).wait()

    def compute(slot):
        for g in range(B // L):
            rows = lax.iota(I32, L) + g * L
            asv = plsc.load_gather(hrows,
                                   [jnp.full((L,), slot, I32), rows,
                                    jnp.full((L,), D + 1, I32)])
            adv = adbuf[slot, pl.ds(g * L, L)]
            s = asv + adv
            wbuf[pl.ds(L, L)] = jnp.exp(jnp.maximum(s, 0.2 * s))
            for r in range(L):
                wr = plsc.load_gather(wbuf, [jnp.full((L,), L + r, I32)])
                e = g * L + r
                for k in range(DEXT // L):
                    hrows[slot, e, pl.ds(k * L, L)] = (
                        hrows[slot, e, pl.ds(k * L, L)] * wr)

    fetch(0, 0)
    fetch(1, 1)

    def body(i, carry):
        for u in range(NSLOT):
            jb = NSLOT * i + u
            p = u
            q = (u + 2) % NSLOT
            wait_gather(p)
            compute(p)
            scatter(p)
            if u == 0:
                @pl.when(i > 0)
                def _():
                    wait_scatter(q)
            else:
                wait_scatter(q)

            @pl.when(jb + 2 < nbv)
            def _():
                fetch(jb + 2, q)
        return carry

    lax.fori_loop(0, nbv // NSLOT, body, 0)
    wait_scatter(NSLOT - 1)
    plsc.subcore_barrier()

    pltpu.sync_copy(acc.at[pl.ds(base, rpt)], out.at[cid, pl.ds(base, rpt)])


def _sc_edge(hext, adt, sds, nb0, nb1, n):
    mesh = plsc.VectorSubcoreMesh(
        core_axis_name="c", subcore_axis_name="s", num_cores=NC,
        num_subcores=NS)
    return pl.kernel(
        functools.partial(_sc_body, nb0, nb1, n),
        out_type=jax.ShapeDtypeStruct((NC, APAD, DEXT), F32),
        mesh=mesh,
        compiler_params=pltpu.CompilerParams(
            use_tc_tiling_on_sc=False, needs_layout_passes=False),
        scratch_types=[
            pltpu.VMEM_SHARED((APAD, DEXT), F32),
            pltpu.VMEM((NSLOT, 2, B), I32),
            pltpu.VMEM((NSLOT, B, DEXT), F32),
            pltpu.VMEM((NSLOT, B), F32),
            pltpu.VMEM((2 * L,), F32),
            pltpu.SemaphoreType.DMA,
            pltpu.SemaphoreType.DMA,
            pltpu.SemaphoreType.DMA,
        ],
    )(hext, adt, sds)


def _post_body(n, x_ref, hext_ref, acc_ref, as_ref, ad_ref, b_ref, g_ref,
               beta_ref, out_ref):
    x = x_ref[...]
    h = hext_ref[pl.ds(0, n), pl.ds(0, D)]
    num = (acc_ref[0, pl.ds(0, n), pl.ds(0, D)]
           + acc_ref[1, pl.ds(0, n), pl.ds(0, D)])
    dent = (acc_ref[0, pl.ds(0, n), pl.ds(D, 16)]
            + acc_ref[1, pl.ds(0, n), pl.ds(D, 16)])
    den = dent[:, 0:1]
    a_s = jnp.sum(h * as_ref[...], axis=1, keepdims=True)
    a_d = jnp.sum(h * ad_ref[...], axis=1, keepdims=True)
    s = a_s + a_d
    w_self = jnp.exp(jnp.maximum(s, 0.2 * s))
    x_att = (num + w_self * h) / (den + w_self + 1e-16) + b_ref[...]
    x2 = x + x_att
    mu = jnp.mean(x2, axis=1, keepdims=True)
    var = jnp.mean((x2 - mu) ** 2, axis=1, keepdims=True)
    xn = (x2 - mu) * lax.rsqrt(var + 1e-5) * g_ref[...] + beta_ref[...]
    out_ref[...] = jnp.maximum(xn, 0.0)


def _tc_post(x, hext, acc, a_src, a_dst, b, g, beta):
    n = x.shape[0]
    return pl.pallas_call(
        functools.partial(_post_body, n),
        out_shape=jax.ShapeDtypeStruct((n, D), F32),
    )(x, hext, acc, a_src.reshape(1, D), a_dst.reshape(1, D),
      b.reshape(1, D), g.reshape(1, D), beta.reshape(1, D))


def kernel(x, edge_index, W0, a_src0, a_dst0, b0, g0, beta0,
           W1, a_src1, a_dst1, b1, g1, beta1):
    n = x.shape[0]
    e = edge_index.shape[1]
    nbsum = max(2 * NSLOT, _round_up(-(-e // (NS * B)), NSLOT))
    nb0 = max(NSLOT, _round_up(int(nbsum * SPLIT0), NSLOT))
    nb1 = _round_up(nbsum - nb0, NSLOT)
    nbmax = max(nb0, nb1)

    src = edge_index[0].astype(I32)
    dst = edge_index[1].astype(I32)
    cap = NS * (nb0 + nb1) * B
    srcp = jnp.concatenate([src, jnp.full((cap - e,), n, I32)])
    dstp = jnp.concatenate([dst, jnp.full((cap - e,), n, I32)])
    nbw = [nb0 if w % NC == 0 else nb1 for w in range(NW)]
    offs = [0]
    for c in nbw:
        offs.append(offs[-1] + c * B)
    slabs = []
    for w in range(NW):
        cnt = nbw[w] * B
        tailpad = jnp.full(((nbmax - nbw[w]) * B,), n, I32)
        seg_s = jnp.concatenate([srcp[offs[w]:offs[w] + cnt], tailpad])
        seg_d = jnp.concatenate([dstp[offs[w]:offs[w] + cnt], tailpad])
        slabs.append(jnp.stack([seg_s.reshape(nbmax, B),
                                seg_d.reshape(nbmax, B)], axis=1))
    sds = jnp.stack(slabs)

    for (W, a_s, a_d, b, g, beta) in (
            (W0, a_src0, a_dst0, b0, g0, beta0),
            (W1, a_src1, a_dst1, b1, g1, beta1)):
        hext, adt = _tc_pre(x, W, a_s, a_d)
        acc = _sc_edge(hext, adt.reshape(HPAD), sds, nb0, nb1, n)
        x = _tc_post(x, hext, acc, a_s, a_d, b, g, beta)
    return x


# speedup vs baseline: 1.2400x; 1.0017x over previous
import functools
import jax
import jax.numpy as jnp
from jax import lax
from jax.experimental import pallas as pl
from jax.experimental.pallas import tpu as pltpu
from jax.experimental.pallas import tpu_sc as plsc

F32 = jnp.float32
I32 = jnp.int32

D = 128
DEXT = 144
NC, NS, L = 2, 16, 16
NW = NC * NS
B = 80
NSLOT = 3
HPAD = 10240
APAD = 10112


def _round_up(a, m):
    return (a + m - 1) // m * m


def _pre_body(n, x_ref, w_ref, as_ref, ad_ref, hext_ref, adt_ref):
    h = jnp.dot(x_ref[...], w_ref[...].T, preferred_element_type=F32)
    a_s = jnp.sum(h * as_ref[...], axis=1, keepdims=True)
    a_d = jnp.sum(h * ad_ref[...], axis=1)
    hext_ref[pl.ds(0, n), pl.ds(0, D)] = h
    col16 = lax.broadcasted_iota(I32, (n, 16), 1)
    tail = jnp.where(col16 == 0, 1.0, jnp.where(col16 == 1, a_s, 0.0))
    hext_ref[pl.ds(0, n), pl.ds(D, 16)] = tail.astype(F32)
    hext_ref[pl.ds(n, HPAD - n), :] = jnp.zeros((HPAD - n, DEXT), F32)
    adt_ref[0, pl.ds(0, n)] = a_d
    adt_ref[0, pl.ds(n, HPAD - n)] = jnp.zeros((HPAD - n,), F32)


def _tc_pre(x, W, a_src, a_dst):
    n = x.shape[0]
    return pl.pallas_call(
        functools.partial(_pre_body, n),
        out_shape=(
            jax.ShapeDtypeStruct((HPAD, DEXT), F32),
            jax.ShapeDtypeStruct((1, HPAD), F32),
        ),
    )(x, W, a_src.reshape(1, D), a_dst.reshape(1, D))


def _sc_body(nbv, n, hext, adt, sds, out, acc, idx, hrows, adbuf, wbuf,
             ghsem, gasem, ssem):
    cid = lax.axis_index("c")
    sid = lax.axis_index("s")
    wid = sid * NC + cid
    rpt = APAD // NS
    base = sid * rpt

    pltpu.sync_copy(hext.at[pl.ds(HPAD - B, B)], hrows.at[0])
    nfull, rem = rpt // B, rpt % B
    for j in range(nfull):
        pltpu.sync_copy(hrows.at[0], acc.at[pl.ds(base + j * B, B)])
    if rem:
        pltpu.sync_copy(hrows.at[0, pl.ds(0, rem)],
                        acc.at[pl.ds(base + nfull * B, rem)])
    plsc.subcore_barrier()

    def fetch(jb, slot):
        pltpu.sync_copy(sds.at[wid, jb], idx.at[slot])
        pltpu.async_copy(hext.at[idx.at[slot, 0]], hrows.at[slot], ghsem)
        pltpu.async_copy(adt.at[idx.at[slot, 1]], adbuf.at[slot], gasem)

    def wait_gather(slot):
        pltpu.make_async_copy(hext.at[idx.at[slot, 0]], hrows.at[slot],
                              ghsem).wait()
        pltpu.make_async_copy(adt.at[idx.at[slot, 1]], adbuf.at[slot],
                              gasem).wait()

    def scatter(slot):
        pltpu.async_copy(hrows.at[slot], acc.at[idx.at[slot, 1]], ssem,
                         add=True)

    def wait_scatter(slot):
        pltpu.make_async_copy(hrows.at[slot], acc.at[idx.at[slot, 1]],
                              ssem).wait()

    def compute(slot):
        for g in range(B // L):
            rows = lax.iota(I32, L) + g * L
            asv = plsc.load_gather(hrows,
                                   [jnp.full((L,), slot, I32), rows,
                                    jnp.full((L,), D + 1, I32)])
            adv = adbuf[slot, pl.ds(g * L, L)]
            s = asv + adv
            wbuf[pl.ds(L, L)] = jnp.exp(jnp.maximum(s, 0.2 * s))
            for r in range(L):
                wr = plsc.load_gather(wbuf, [jnp.full((L,), L + r, I32)])
                e = g * L + r
                for k in range(DEXT // L):
                    hrows[slot, e, pl.ds(k * L, L)] = (
                        hrows[slot, e, pl.ds(k * L, L)] * wr)

    fetch(0, 0)
    fetch(1, 1)

    def body(i, carry):
        for u in range(NSLOT):
            jb = NSLOT * i + u
            p = u
            q = (u + 2) % NSLOT
            wait_gather(p)
            compute(p)
            scatter(p)
            if u == 0:
                @pl.when(i > 0)
                def _():
                    wait_scatter(q)
            else:
                wait_scatter(q)

            @pl.when(jb + 2 < nbv)
            def _():
                fetch(jb + 2, q)
        return carry

    lax.fori_loop(0, nbv // NSLOT, body, 0)
    wait_scatter(NSLOT - 1)
    plsc.subcore_barrier()

    pltpu.sync_copy(acc.at[pl.ds(base, rpt)], out.at[cid, pl.ds(base, rpt)])


def _sc_edge(hext, adt, sds, nb, n):
    mesh = plsc.VectorSubcoreMesh(
        core_axis_name="c", subcore_axis_name="s", num_cores=NC,
        num_subcores=NS)
    return pl.kernel(
        functools.partial(_sc_body, nb, n),
        out_type=jax.ShapeDtypeStruct((NC, APAD, DEXT), F32),
        mesh=mesh,
        compiler_params=pltpu.CompilerParams(
            use_tc_tiling_on_sc=False, needs_layout_passes=False),
        scratch_types=[
            pltpu.VMEM_SHARED((APAD, DEXT), F32),
            pltpu.VMEM((NSLOT, 2, B), I32),
            pltpu.VMEM((NSLOT, B, DEXT), F32),
            pltpu.VMEM((NSLOT, B), F32),
            pltpu.VMEM((2 * L,), F32),
            pltpu.SemaphoreType.DMA,
            pltpu.SemaphoreType.DMA,
            pltpu.SemaphoreType.DMA,
        ],
    )(hext, adt, sds)


def _post_body(n, x_ref, hext_ref, acc_ref, as_ref, ad_ref, b_ref, g_ref,
               beta_ref, out_ref):
    x = x_ref[...]
    h = hext_ref[pl.ds(0, n), pl.ds(0, D)]
    num = (acc_ref[0, pl.ds(0, n), pl.ds(0, D)]
           + acc_ref[1, pl.ds(0, n), pl.ds(0, D)])
    dent = (acc_ref[0, pl.ds(0, n), pl.ds(D, 16)]
            + acc_ref[1, pl.ds(0, n), pl.ds(D, 16)])
    den = dent[:, 0:1]
    a_s = jnp.sum(h * as_ref[...], axis=1, keepdims=True)
    a_d = jnp.sum(h * ad_ref[...], axis=1, keepdims=True)
    s = a_s + a_d
    w_self = jnp.exp(jnp.maximum(s, 0.2 * s))
    x_att = (num + w_self * h) / (den + w_self + 1e-16) + b_ref[...]
    x2 = x + x_att
    mu = jnp.mean(x2, axis=1, keepdims=True)
    var = jnp.mean((x2 - mu) ** 2, axis=1, keepdims=True)
    xn = (x2 - mu) * lax.rsqrt(var + 1e-5) * g_ref[...] + beta_ref[...]
    out_ref[...] = jnp.maximum(xn, 0.0)


def _tc_post(x, hext, acc, a_src, a_dst, b, g, beta):
    n = x.shape[0]
    return pl.pallas_call(
        functools.partial(_post_body, n),
        out_shape=jax.ShapeDtypeStruct((n, D), F32),
    )(x, hext, acc, a_src.reshape(1, D), a_dst.reshape(1, D),
      b.reshape(1, D), g.reshape(1, D), beta.reshape(1, D))


def kernel(x, edge_index, W0, a_src0, a_dst0, b0, g0, beta0,
           W1, a_src1, a_dst1, b1, g1, beta1):
    n = x.shape[0]
    e = edge_index.shape[1]
    nb = max(NSLOT, _round_up(-(-e // (NW * B)), NSLOT))
    cap = NW * nb * B

    src = edge_index[0].astype(I32)
    dst = edge_index[1].astype(I32)
    pad = jnp.full((cap - e,), n, I32)
    srcs = jnp.concatenate([src, pad]).reshape(NW, nb, 1, B)
    dsts = jnp.concatenate([dst, pad]).reshape(NW, nb, 1, B)
    sds = jnp.concatenate([srcs, dsts], axis=2)

    for (W, a_s, a_d, b, g, beta) in (
            (W0, a_src0, a_dst0, b0, g0, beta0),
            (W1, a_src1, a_dst1, b1, g1, beta1)):
        hext, adt = _tc_pre(x, W, a_s, a_d)
        acc = _sc_edge(hext, adt.reshape(HPAD), sds, nb, n)
        x = _tc_post(x, hext, acc, a_s, a_d, b, g, beta)
    return x


# speedup vs baseline: 1.2435x; 1.0028x over previous
import functools
import jax
import jax.numpy as jnp
from jax import lax
from jax.experimental import pallas as pl
from jax.experimental.pallas import tpu as pltpu
from jax.experimental.pallas import tpu_sc as plsc

F32 = jnp.float32
I32 = jnp.int32

D = 128
DEXT = 144
NC, NS, L = 2, 16, 16
NW = NC * NS
B = 80
NSLOT = 3
HPAD = 10240
APAD = 10112


def _round_up(a, m):
    return (a + m - 1) // m * m


def _pre_body(n, x_ref, w_ref, as_ref, ad_ref, hext_ref, adt_ref):
    h = jnp.dot(x_ref[...], w_ref[...].T, preferred_element_type=F32)
    a_s = jnp.sum(h * as_ref[...], axis=1, keepdims=True)
    a_d = jnp.sum(h * ad_ref[...], axis=1)
    hext_ref[pl.ds(0, n), pl.ds(0, D)] = h
    col16 = lax.broadcasted_iota(I32, (n, 16), 1)
    tail = jnp.where(col16 == 0, 1.0, jnp.where(col16 == 1, a_s, 0.0))
    hext_ref[pl.ds(0, n), pl.ds(D, 16)] = tail.astype(F32)
    hext_ref[pl.ds(n, HPAD - n), :] = jnp.zeros((HPAD - n, DEXT), F32)
    adt_ref[0, pl.ds(0, n)] = a_d
    adt_ref[0, pl.ds(n, HPAD - n)] = jnp.zeros((HPAD - n,), F32)


def _tc_pre(x, W, a_src, a_dst):
    n = x.shape[0]
    return pl.pallas_call(
        functools.partial(_pre_body, n),
        out_shape=(
            jax.ShapeDtypeStruct((HPAD, DEXT), F32),
            jax.ShapeDtypeStruct((1, HPAD), F32),
        ),
    )(x, W, a_src.reshape(1, D), a_dst.reshape(1, D))


def _sc_body(nbv, n, hext, adt, sds, out, acc, idx, hrows, adbuf, wbuf,
             ghsem, gasem, ssem):
    cid = lax.axis_index("c")
    sid = lax.axis_index("s")
    wid = sid * NC + cid
    rpt = APAD // NS
    base = sid * rpt

    pltpu.sync_copy(hext.at[pl.ds(HPAD - B, B)], hrows.at[0])
    nfull, rem = rpt // B, rpt % B
    for j in range(nfull):
        pltpu.sync_copy(hrows.at[0], acc.at[pl.ds(base + j * B, B)])
    if rem:
        pltpu.sync_copy(hrows.at[0, pl.ds(0, rem)],
                        acc.at[pl.ds(base + nfull * B, rem)])
    plsc.subcore_barrier()

    def fetch(jb, slot):
        pltpu.sync_copy(sds.at[wid, jb], idx.at[slot])
        pltpu.async_copy(hext.at[idx.at[slot, 0]], hrows.at[slot], ghsem)
        pltpu.async_copy(adt.at[idx.at[slot, 1]], adbuf.at[slot], gasem)

    def wait_gather(slot):
        pltpu.make_async_copy(hext.at[idx.at[slot, 0]], hrows.at[slot],
                              ghsem).wait()
        pltpu.make_async_copy(adt.at[idx.at[slot, 1]], adbuf.at[slot],
                              gasem).wait()

    def scatter(slot):
        pltpu.async_copy(hrows.at[slot], acc.at[idx.at[slot, 1]], ssem,
                         add=True)

    def wait_scatter(slot):
        pltpu.make_async_copy(hrows.at[slot], acc.at[idx.at[slot, 1]],
                              ssem).wait()

    def compute(slot):
        for g in range(B // L):
            rows = lax.iota(I32, L) + g * L
            asv = plsc.load_gather(hrows,
                                   [jnp.full((L,), slot, I32), rows,
                                    jnp.full((L,), D + 1, I32)])
            adv = adbuf[slot, pl.ds(g * L, L)]
            s = asv + adv
            wbuf[pl.ds(L, L)] = jnp.exp(jnp.maximum(s, 0.2 * s))
            for r in range(L):
                wr = plsc.load_gather(wbuf, [jnp.full((L,), L + r, I32)])
                e = g * L + r
                for k in range(DEXT // L):
                    hrows[slot, e, pl.ds(k * L, L)] = (
                        hrows[slot, e, pl.ds(k * L, L)] * wr)

    fetch(0, 0)
    fetch(1, 1)

    def body(i, carry):
        for u in range(NSLOT):
            jb = NSLOT * i + u
            p = u
            q = (u + 2) % NSLOT
            wait_gather(p)
            compute(p)
            scatter(p)
            if u == 0:
                @pl.when(i > 0)
                def _():
                    wait_scatter(q)
            else:
                wait_scatter(q)

            @pl.when(jb + 2 < nbv)
            def _():
                fetch(jb + 2, q)
        return carry

    lax.fori_loop(0, nbv // NSLOT, body, 0)
    wait_scatter(NSLOT - 1)
    plsc.subcore_barrier()

    pltpu.sync_copy(acc.at[pl.ds(base, rpt)], out.at[cid, pl.ds(base, rpt)])


def _sc_edge(hext, adt, sds, nb, n):
    mesh = plsc.VectorSubcoreMesh(
        core_axis_name="c", subcore_axis_name="s", num_cores=NC,
        num_subcores=NS)
    return pl.kernel(
        functools.partial(_sc_body, nb, n),
        out_type=jax.ShapeDtypeStruct((NC, APAD, DEXT), F32),
        mesh=mesh,
        compiler_params=pltpu.CompilerParams(
            use_tc_tiling_on_sc=False, needs_layout_passes=False),
        scratch_types=[
            pltpu.VMEM_SHARED((APAD, DEXT), F32),
            pltpu.VMEM((NSLOT, 2, B), I32),
            pltpu.VMEM((NSLOT, B, DEXT), F32),
            pltpu.VMEM((NSLOT, B), F32),
            pltpu.VMEM((2 * L,), F32),
            pltpu.SemaphoreType.DMA,
            pltpu.SemaphoreType.DMA,
            pltpu.SemaphoreType.DMA,
        ],
    )(hext, adt, sds)


def _post_body(n, x_ref, hext_ref, acc_ref, as_ref, ad_ref, b_ref, g_ref,
               beta_ref, out_ref):
    x = x_ref[...]
    h = hext_ref[pl.ds(0, n), pl.ds(0, D)]
    num = (acc_ref[0, pl.ds(0, n), pl.ds(0, D)]
           + acc_ref[1, pl.ds(0, n), pl.ds(0, D)])
    dent = (acc_ref[0, pl.ds(0, n), pl.ds(D, 16)]
            + acc_ref[1, pl.ds(0, n), pl.ds(D, 16)])
    den = dent[:, 0:1]
    a_s = jnp.sum(h * as_ref[...], axis=1, keepdims=True)
    a_d = jnp.sum(h * ad_ref[...], axis=1, keepdims=True)
    s = a_s + a_d
    w_self = jnp.exp(jnp.maximum(s, 0.2 * s))
    x_att = (num + w_self * h) / (den + w_self + 1e-16) + b_ref[...]
    x2 = x + x_att
    mu = jnp.mean(x2, axis=1, keepdims=True)
    var = jnp.mean((x2 - mu) ** 2, axis=1, keepdims=True)
    xn = (x2 - mu) * lax.rsqrt(var + 1e-5) * g_ref[...] + beta_ref[...]
    out_ref[...] = jnp.maximum(xn, 0.0)


def _tc_post(x, hext, acc, a_src, a_dst, b, g, beta):
    n = x.shape[0]
    return pl.pallas_call(
        functools.partial(_post_body, n),
        out_shape=jax.ShapeDtypeStruct((n, D), F32),
    )(x, hext, acc, a_src.reshape(1, D), a_dst.reshape(1, D),
      b.reshape(1, D), g.reshape(1, D), beta.reshape(1, D))


def _mid_body(n, x_ref, hext0_ref, acc_ref, as0_ref, ad0_ref, b_ref, g_ref,
              beta_ref, w1_ref, as1_ref, ad1_ref, hext1_ref, adt1_ref,
              x1_ref):
    x = x_ref[...]
    h = hext0_ref[pl.ds(0, n), pl.ds(0, D)]
    num = (acc_ref[0, pl.ds(0, n), pl.ds(0, D)]
           + acc_ref[1, pl.ds(0, n), pl.ds(0, D)])
    dent = (acc_ref[0, pl.ds(0, n), pl.ds(D, 16)]
            + acc_ref[1, pl.ds(0, n), pl.ds(D, 16)])
    den = dent[:, 0:1]
    a_s = jnp.sum(h * as0_ref[...], axis=1, keepdims=True)
    a_d = jnp.sum(h * ad0_ref[...], axis=1, keepdims=True)
    s = a_s + a_d
    w_self = jnp.exp(jnp.maximum(s, 0.2 * s))
    x_att = (num + w_self * h) / (den + w_self + 1e-16) + b_ref[...]
    x2 = x + x_att
    mu = jnp.mean(x2, axis=1, keepdims=True)
    var = jnp.mean((x2 - mu) ** 2, axis=1, keepdims=True)
    xn = (x2 - mu) * lax.rsqrt(var + 1e-5) * g_ref[...] + beta_ref[...]
    x1 = jnp.maximum(xn, 0.0)
    x1_ref[...] = x1

    h1 = jnp.dot(x1, w1_ref[...].T, preferred_element_type=F32)
    a_s1 = jnp.sum(h1 * as1_ref[...], axis=1, keepdims=True)
    a_d1 = jnp.sum(h1 * ad1_ref[...], axis=1)
    hext1_ref[pl.ds(0, n), pl.ds(0, D)] = h1
    col16 = lax.broadcasted_iota(I32, (n, 16), 1)
    tail = jnp.where(col16 == 0, 1.0, jnp.where(col16 == 1, a_s1, 0.0))
    hext1_ref[pl.ds(0, n), pl.ds(D, 16)] = tail.astype(F32)
    hext1_ref[pl.ds(n, HPAD - n), :] = jnp.zeros((HPAD - n, DEXT), F32)
    adt1_ref[0, pl.ds(0, n)] = a_d1
    adt1_ref[0, pl.ds(n, HPAD - n)] = jnp.zeros((HPAD - n,), F32)


def _tc_mid(x, hext0, acc, a_s0, a_d0, b0, g0, beta0, W1, a_s1, a_d1):
    n = x.shape[0]
    return pl.pallas_call(
        functools.partial(_mid_body, n),
        compiler_params=pltpu.CompilerParams(
            vmem_limit_bytes=100 * 1024 * 1024),
        out_shape=(
            jax.ShapeDtypeStruct((HPAD, DEXT), F32),
            jax.ShapeDtypeStruct((1, HPAD), F32),
            jax.ShapeDtypeStruct((n, D), F32),
        ),
    )(x, hext0, acc, a_s0.reshape(1, D), a_d0.reshape(1, D),
      b0.reshape(1, D), g0.reshape(1, D), beta0.reshape(1, D),
      W1, a_s1.reshape(1, D), a_d1.reshape(1, D))


def kernel(x, edge_index, W0, a_src0, a_dst0, b0, g0, beta0,
           W1, a_src1, a_dst1, b1, g1, beta1):
    n = x.shape[0]
    e = edge_index.shape[1]
    nb = max(NSLOT, _round_up(-(-e // (NW * B)), NSLOT))
    cap = NW * nb * B

    src = edge_index[0].astype(I32)
    dst = edge_index[1].astype(I32)
    pad = jnp.full((cap - e,), n, I32)
    srcs = jnp.concatenate([src, pad]).reshape(NW, nb, 1, B)
    dsts = jnp.concatenate([dst, pad]).reshape(NW, nb, 1, B)
    sds = jnp.concatenate([srcs, dsts], axis=2)

    hext0, adt0 = _tc_pre(x, W0, a_src0, a_dst0)
    acc0 = _sc_edge(hext0, adt0.reshape(HPAD), sds, nb, n)
    hext1, adt1, x1 = _tc_mid(x, hext0, acc0, a_src0, a_dst0, b0, g0, beta0,
                              W1, a_src1, a_dst1)
    acc1 = _sc_edge(hext1, adt1.reshape(HPAD), sds, nb, n)
    return _tc_post(x1, hext1, acc1, a_src1, a_dst1, b1, g1, beta1)
